# baseline (speedup 1.0000x reference)
import functools

import jax
import jax.numpy as jnp
from jax import lax
from jax.experimental import pallas as pl
from jax.experimental.pallas import tpu as pltpu
from jax.experimental.pallas import tpu_sc as plsc

NUM_SAMPLED = 255
S_PAD = 256
_CHUNK = 128


def _sampled_consts(V):
    logv1 = jnp.log(jnp.float32(V) + 1.0)
    skey = jax.random.fold_in(jax.random.key(0), 12345)
    u = jax.random.uniform(skey, (NUM_SAMPLED,), dtype=jnp.float32)
    s = jnp.floor(jnp.exp(u * logv1)) - 1.0
    sampled = jnp.clip(s, 0, V - 1).astype(jnp.int32)
    cs = sampled.astype(jnp.float32)
    p_samp = (jnp.log(cs + 2.0) - jnp.log(cs + 1.0)) / logv1
    nlse = -jnp.log(-jnp.expm1(NUM_SAMPLED * jnp.log1p(-p_samp)))
    sampled_pad = jnp.concatenate([sampled, jnp.zeros((1,), jnp.int32)])
    nlse_pad = jnp.concatenate([nlse, jnp.full((1,), -1e30, jnp.float32)])
    cmp_pad = jnp.concatenate([sampled, jnp.full((1,), -1, jnp.int32)])
    return sampled_pad, nlse_pad, cmp_pad


@functools.lru_cache(maxsize=None)
def _make_sc_gather(R, B, S):
    info = plsc.get_sparse_core_info()
    NC, NS = info.num_cores, info.num_subcores
    NW = NC * NS
    n_chunks = (B + S) // _CHUNK
    n_extra = n_chunks - NW
    assert 0 <= n_extra <= NW

    mesh = plsc.VectorSubcoreMesh(core_axis_name="c", subcore_axis_name="s")

    @functools.partial(
        pl.kernel,
        mesh=mesh,
        out_type=jax.ShapeDtypeStruct((B + S, _CHUNK), jnp.float32),
        scratch_types=[
            pltpu.VMEM((_CHUNK,), jnp.int32),
            pltpu.VMEM((_CHUNK, _CHUNK), jnp.float32),
            pltpu.SemaphoreType.DMA,
        ],
        compiler_params=pltpu.CompilerParams(use_tc_tiling_on_sc=True),
    )
    def gather(table_hbm, sup_hbm, out_hbm, idx_v, rows_v, sem):
        wid = lax.axis_index("s") * NC + lax.axis_index("c")

        def do_chunk(c):
            base = pl.multiple_of(c * _CHUNK, _CHUNK)
            pltpu.sync_copy(sup_hbm.at[pl.ds(base, _CHUNK)], idx_v)
            pltpu.async_copy(table_hbm.at[idx_v], rows_v, sem).wait()
            pltpu.sync_copy(rows_v, out_hbm.at[pl.ds(base, _CHUNK), :])

        do_chunk(wid)
        if n_extra:
            @pl.when(wid < n_extra)
            def _():
                do_chunk(NW + wid)

    return gather


def _dense_body(logv1_ref, idx_ref, grp_ref, u_ref, trow_ref, srow_ref,
                sgrp_ref, cmp_ref, nlse_ref, out_ref):
    BB, D = u_ref.shape

    trow = trow_ref[...]
    grp = grp_ref[...]
    sel = jnp.zeros((BB, D), jnp.float32)
    srow = srow_ref[...]
    sgrp = sgrp_ref[...]
    sw = jnp.zeros((S_PAD, D), jnp.float32)
    for g in range(128 // D):
        sel = sel + trow[:, g * D:(g + 1) * D] * (grp == g).astype(jnp.float32)
        sw = sw + srow[:, g * D:(g + 1) * D] * (sgrp == g).astype(jnp.float32)

    logv1 = logv1_ref[0]
    u = u_ref[...]
    c = idx_ref[...].astype(jnp.float32)
    p_true = (jnp.log(c + 2.0) - jnp.log(c + 1.0)) / logv1
    q = NUM_SAMPLED * jnp.log1p(-p_true)
    lte_small = jnp.log(-q) + jnp.log1p(
        q * (0.5 + q * (1.0 / 6.0 + q * (1.0 / 24.0))))
    lte_big = jnp.log(1.0 - jnp.exp(q))
    lte = jnp.where(q > -0.1, lte_small, lte_big)
    tl = jnp.sum(u * sel, axis=1, keepdims=True) - lte

    slt = lax.dot_general(
        u, sw, (((1,), (1,)), ((), ())),
        preferred_element_type=jnp.float32,
    )
    slt = slt + nlse_ref[...]
    acc = (cmp_ref[...] == idx_ref[...]).astype(jnp.float32)
    slt = slt - acc * 1e9

    m = jnp.maximum(jnp.max(slt, axis=1, keepdims=True), tl)
    z = jnp.sum(jnp.exp(slt - m), axis=1, keepdims=True) + jnp.exp(tl - m)
    out_ref[...] = jnp.log(z) + m - tl


def _dense(logv1, idxc, grpc, user, gat, sgrp, cmps, nlses, *, interpret=False):
    B, D = user.shape
    BB = 512
    grid = (B // BB,)
    s_block = B // S_PAD
    return pl.pallas_call(
        _dense_body,
        grid=grid,
        in_specs=[
            pl.BlockSpec(memory_space=pltpu.SMEM),
            pl.BlockSpec((BB, 1), lambda i: (i, 0)),
            pl.BlockSpec((BB, 1), lambda i: (i, 0)),
            pl.BlockSpec((BB, D), lambda i: (i, 0)),
            pl.BlockSpec((BB, 128), lambda i: (i, 0)),
            pl.BlockSpec((S_PAD, 128), lambda i: (s_block, 0)),
            pl.BlockSpec((S_PAD, 1), lambda i: (0, 0)),
            pl.BlockSpec((1, S_PAD), lambda i: (0, 0)),
            pl.BlockSpec((1, S_PAD), lambda i: (0, 0)),
        ],
        out_specs=pl.BlockSpec((BB, 1), lambda i: (i, 0)),
        out_shape=jax.ShapeDtypeStruct((B, 1), jnp.float32),
        interpret=interpret,
    )(logv1, idxc, grpc, user, gat, gat, sgrp, cmps, nlses)


def kernel(item_embeddings, user_vec, item_idx, zero_bias):
    V, D = item_embeddings.shape
    B = user_vec.shape[0]
    GPS = 128 // D
    idx = item_idx.reshape(-1).astype(jnp.int32)

    logv1 = jnp.log(jnp.float32(V) + 1.0)
    sampled_pad, nlse_pad, cmp_pad = _sampled_consts(V)

    sup = jnp.concatenate([idx, sampled_pad]) // GPS

    gat = _make_sc_gather(V // GPS, B, S_PAD)(
        item_embeddings.reshape(V // GPS, D * GPS), sup
    )

    loss = _dense(
        logv1.reshape(1),
        idx.reshape(B, 1),
        (idx % GPS).reshape(B, 1),
        user_vec,
        gat,
        (sampled_pad % GPS).reshape(S_PAD, 1),
        cmp_pad.reshape(1, S_PAD),
        nlse_pad.reshape(1, S_PAD),
    )
    return loss
